# baseline (speedup 1.0000x reference)
import math

import jax
import jax.numpy as jnp
from jax.experimental import pallas as pl
from jax.experimental.pallas import tpu as pltpu

_N_BASIS = 64
_OUT_SIZE = 32
_GROUP = 4

_INV_2PI = 0.15915494309189535
_TWO_PI = 6.283185307179586
_C0 = 0.9999991998413438
_C1 = -0.49999415816713466
_C2 = 0.04165973316165389
_C3 = -0.0013858663490020644
_C4 = 2.4201479340302904e-05
_C5 = -2.1967044652900134e-07


def _round_up(a, b):
    return -(-a // b) * b


def _poly_cos(u):
    n = jnp.round(u * _INV_2PI)
    r = u - n * _TWO_PI
    s = r * r
    p = _C5 * s + _C4
    p = p * s + _C3
    p = p * s + _C2
    p = p * s + _C1
    return p * s + _C0


def _fast_kernel(x_ref, s_ref, w_ref, b_ref, o_ref):
    b_rows, d1 = o_ref.shape[0], o_ref.shape[1]
    u = jax.lax.dot_general(
        x_ref[0], s_ref[...], (((0,), (0,)), ((), ())),
        preferred_element_type=jnp.float32)
    emb = _poly_cos(u)
    out = jnp.dot(emb, w_ref[...], preferred_element_type=jnp.float32)
    out = out + b_ref[...]
    pieces = [
        out[:, 32 * g:32 * (g + 1)].reshape(b_rows // _GROUP, d1, _OUT_SIZE)
        for g in range(_GROUP)
    ]
    o_ref[...] = jnp.concatenate(pieces, axis=0)


def _generic_kernel(x_ref, s_ref, w_ref, b_ref, o_ref):
    u = jax.lax.dot_general(
        x_ref[...], s_ref[...], (((0,), (0,)), ((), ())),
        preferred_element_type=jnp.float32)
    emb = _poly_cos(u)
    out = jnp.dot(emb, w_ref[...], preferred_element_type=jnp.float32)
    o_ref[...] = out + b_ref[...]


def _block_diag_mats(weight, bias):
    ks = jnp.arange(1, _N_BASIS + 1, dtype=jnp.float32) * math.pi
    eye_g = jnp.eye(_GROUP, dtype=jnp.float32)
    s_mat = jnp.kron(eye_g, ks[None, :])
    w2 = jnp.kron(eye_g, weight.astype(jnp.float32).T)
    b2 = jnp.tile(bias.astype(jnp.float32), _GROUP).reshape(1, 128)
    return s_mat, w2, b2


def _fast_2d(x, weight, bias):
    d0, d1 = x.shape
    b_rows = _GROUP
    for cand in (8, 16):
        if d0 % cand == 0 and cand * d1 <= 16384:
            b_rows = cand
            break
    tq = b_rows * d1 // _GROUP
    s_mat, w2, b2 = _block_diag_mats(weight, bias)
    x4 = x.astype(jnp.float32).reshape(d0 // b_rows, _GROUP, tq)
    return pl.pallas_call(
        _fast_kernel,
        out_shape=jax.ShapeDtypeStruct((d0, d1, _OUT_SIZE), jnp.float32),
        grid=(d0 // b_rows,),
        in_specs=[
            pl.BlockSpec((1, _GROUP, tq), lambda i: (i, 0, 0)),
            pl.BlockSpec((_GROUP, _GROUP * _N_BASIS), lambda i: (0, 0)),
            pl.BlockSpec((_GROUP * _N_BASIS, 128), lambda i: (0, 0)),
            pl.BlockSpec((1, 128), lambda i: (0, 0)),
        ],
        out_specs=pl.BlockSpec((b_rows, d1, _OUT_SIZE), lambda i: (i, 0, 0)),
        compiler_params=pltpu.CompilerParams(
            dimension_semantics=("parallel",)),
    )(x4, s_mat, w2, b2)


def _generic(x, weight, bias):
    orig_shape = x.shape
    x_flat = x.reshape(-1).astype(jnp.float32)
    n = x_flat.shape[0]
    row_align = 128 * _GROUP
    n_ceil = _round_up(max(n, 1), row_align)
    tn = min(16384, n_ceil)
    tile_rows = tn // _GROUP
    n_pad = _round_up(max(n, 1), tn)
    rows_pad = n_pad // _GROUP
    s_mat, w2, b2 = _block_diag_mats(weight, bias)
    xt = jnp.pad(x_flat, (0, n_pad - n)).reshape(rows_pad, _GROUP).T
    out2 = pl.pallas_call(
        _generic_kernel,
        out_shape=jax.ShapeDtypeStruct((rows_pad, 128), jnp.float32),
        grid=(n_pad // tn,),
        in_specs=[
            pl.BlockSpec((_GROUP, tile_rows), lambda i: (0, i)),
            pl.BlockSpec((_GROUP, _GROUP * _N_BASIS), lambda i: (0, 0)),
            pl.BlockSpec((_GROUP * _N_BASIS, 128), lambda i: (0, 0)),
            pl.BlockSpec((1, 128), lambda i: (0, 0)),
        ],
        out_specs=pl.BlockSpec((tile_rows, 128), lambda i: (i, 0)),
        compiler_params=pltpu.CompilerParams(
            dimension_semantics=("parallel",)),
    )(xt, s_mat, w2, b2)
    out = out2.reshape(n_pad, _OUT_SIZE)[:n]
    return out.reshape(*orig_shape, _OUT_SIZE)


@jax.jit
def kernel(x, weight, bias):
    if (x.ndim == 2 and x.shape[0] % _GROUP == 0
            and x.shape[1] % 128 == 0 and x.size >= 16384):
        return _fast_2d(x, weight, bias)
    return _generic(x, weight, bias)


# speedup vs baseline: 1.0385x; 1.0385x over previous
import math

import jax
import jax.numpy as jnp
from jax.experimental import pallas as pl
from jax.experimental.pallas import tpu as pltpu

_N_BASIS = 64
_OUT_SIZE = 32
_GROUP = 4

_INV_2PI = 0.15915494309189535
_TWO_PI = 6.283185307179586
_C0 = 0.9999991998413438
_C1 = -0.49999415816713466
_C2 = 0.04165973316165389
_C3 = -0.0013858663490020644
_C4 = 2.4201479340302904e-05
_C5 = -2.1967044652900134e-07


def _round_up(a, b):
    return -(-a // b) * b


def _poly_cos(u):
    n = jnp.round(u * _INV_2PI)
    r = u - n * _TWO_PI
    s = r * r
    p = _C5 * s + _C4
    p = p * s + _C3
    p = p * s + _C2
    p = p * s + _C1
    return p * s + _C0


def _fast_kernel(x_ref, s_ref, w_ref, b_ref, o_ref):
    b_rows, d1 = o_ref.shape[0], o_ref.shape[1]
    u = jax.lax.dot_general(
        x_ref[0], s_ref[...], (((0,), (0,)), ((), ())),
        preferred_element_type=jnp.float32)
    emb = _poly_cos(u)
    out = jnp.dot(emb, w_ref[...], preferred_element_type=jnp.float32)
    out = out + b_ref[...]
    pieces = [
        out[:, 32 * g:32 * (g + 1)].reshape(b_rows // _GROUP, d1, _OUT_SIZE)
        for g in range(_GROUP)
    ]
    o_ref[...] = jnp.concatenate(pieces, axis=0)


def _generic_kernel(x_ref, s_ref, w_ref, b_ref, o_ref):
    u = jax.lax.dot_general(
        x_ref[...], s_ref[...], (((0,), (0,)), ((), ())),
        preferred_element_type=jnp.float32)
    emb = _poly_cos(u)
    out = jnp.dot(emb, w_ref[...], preferred_element_type=jnp.float32)
    o_ref[...] = out + b_ref[...]


def _block_diag_mats(weight, bias):
    ks = jnp.arange(1, _N_BASIS + 1, dtype=jnp.float32) * math.pi
    eye_g = jnp.eye(_GROUP, dtype=jnp.float32)
    s_mat = jnp.kron(eye_g, ks[None, :])
    w2 = jnp.kron(eye_g, weight.astype(jnp.float32).T)
    b2 = jnp.tile(bias.astype(jnp.float32), _GROUP).reshape(1, 128)
    return s_mat, w2, b2


def _fast_2d(x, weight, bias):
    d0, d1 = x.shape
    b_rows = _GROUP
    for cand in (32, 16, 8):
        if d0 % cand == 0 and cand * d1 <= 32768:
            b_rows = cand
            break
    tq = b_rows * d1 // _GROUP
    s_mat, w2, b2 = _block_diag_mats(weight, bias)
    x4 = x.astype(jnp.float32).reshape(d0 // b_rows, _GROUP, tq)
    return pl.pallas_call(
        _fast_kernel,
        out_shape=jax.ShapeDtypeStruct((d0, d1, _OUT_SIZE), jnp.float32),
        grid=(d0 // b_rows,),
        in_specs=[
            pl.BlockSpec((1, _GROUP, tq), lambda i: (i, 0, 0)),
            pl.BlockSpec((_GROUP, _GROUP * _N_BASIS), lambda i: (0, 0)),
            pl.BlockSpec((_GROUP * _N_BASIS, 128), lambda i: (0, 0)),
            pl.BlockSpec((1, 128), lambda i: (0, 0)),
        ],
        out_specs=pl.BlockSpec((b_rows, d1, _OUT_SIZE), lambda i: (i, 0, 0)),
        compiler_params=pltpu.CompilerParams(
            dimension_semantics=("parallel",)),
    )(x4, s_mat, w2, b2)


def _generic(x, weight, bias):
    orig_shape = x.shape
    x_flat = x.reshape(-1).astype(jnp.float32)
    n = x_flat.shape[0]
    row_align = 128 * _GROUP
    n_ceil = _round_up(max(n, 1), row_align)
    tn = min(16384, n_ceil)
    tile_rows = tn // _GROUP
    n_pad = _round_up(max(n, 1), tn)
    rows_pad = n_pad // _GROUP
    s_mat, w2, b2 = _block_diag_mats(weight, bias)
    xt = jnp.pad(x_flat, (0, n_pad - n)).reshape(rows_pad, _GROUP).T
    out2 = pl.pallas_call(
        _generic_kernel,
        out_shape=jax.ShapeDtypeStruct((rows_pad, 128), jnp.float32),
        grid=(n_pad // tn,),
        in_specs=[
            pl.BlockSpec((_GROUP, tile_rows), lambda i: (0, i)),
            pl.BlockSpec((_GROUP, _GROUP * _N_BASIS), lambda i: (0, 0)),
            pl.BlockSpec((_GROUP * _N_BASIS, 128), lambda i: (0, 0)),
            pl.BlockSpec((1, 128), lambda i: (0, 0)),
        ],
        out_specs=pl.BlockSpec((tile_rows, 128), lambda i: (i, 0)),
        compiler_params=pltpu.CompilerParams(
            dimension_semantics=("parallel",)),
    )(xt, s_mat, w2, b2)
    out = out2.reshape(n_pad, _OUT_SIZE)[:n]
    return out.reshape(*orig_shape, _OUT_SIZE)


@jax.jit
def kernel(x, weight, bias):
    if (x.ndim == 2 and x.shape[0] % _GROUP == 0
            and x.shape[1] % 128 == 0 and x.size >= 16384):
        return _fast_2d(x, weight, bias)
    return _generic(x, weight, bias)


# speedup vs baseline: 1.0634x; 1.0240x over previous
import math

import jax
import jax.numpy as jnp
from jax.experimental import pallas as pl
from jax.experimental.pallas import tpu as pltpu

_N_BASIS = 64
_OUT_SIZE = 32
_GROUP = 4

_INV_2PI = 0.15915494309189535
_TWO_PI = 6.283185307179586
_C0 = 0.9999582316201325
_C1 = -0.49978806552054783
_C2 = 0.04149345839543556
_C3 = -0.0013388508753153154
_C4 = 1.8770830927479097e-05


def _round_up(a, b):
    return -(-a // b) * b


def _poly_cos(u):
    n = jnp.round(u * _INV_2PI)
    r = u - n * _TWO_PI
    s = r * r
    p = _C4 * s + _C3
    p = p * s + _C2
    p = p * s + _C1
    return p * s + _C0


def _fast_kernel(x_ref, s_ref, w_ref, b_ref, o_ref):
    b_rows, d1 = o_ref.shape[0], o_ref.shape[1]
    u = jax.lax.dot_general(
        x_ref[0], s_ref[...], (((0,), (0,)), ((), ())),
        preferred_element_type=jnp.float32)
    emb = _poly_cos(u)
    out = jnp.dot(emb, w_ref[...], preferred_element_type=jnp.float32)
    out = out + b_ref[...]
    b4 = b_rows // _GROUP
    for g in range(_GROUP):
        o_ref[g * b4:(g + 1) * b4] = (
            out[:, 32 * g:32 * (g + 1)].reshape(b4, d1, _OUT_SIZE))


def _generic_kernel(x_ref, s_ref, w_ref, b_ref, o_ref):
    u = jax.lax.dot_general(
        x_ref[...], s_ref[...], (((0,), (0,)), ((), ())),
        preferred_element_type=jnp.float32)
    emb = _poly_cos(u)
    out = jnp.dot(emb, w_ref[...], preferred_element_type=jnp.float32)
    o_ref[...] = out + b_ref[...]


def _block_diag_mats(weight, bias):
    ks = jnp.arange(1, _N_BASIS + 1, dtype=jnp.float32) * math.pi
    eye_g = jnp.eye(_GROUP, dtype=jnp.float32)
    s_mat = jnp.kron(eye_g, ks[None, :])
    w2 = jnp.kron(eye_g, weight.astype(jnp.float32).T)
    b2 = jnp.tile(bias.astype(jnp.float32), _GROUP).reshape(1, 128)
    return s_mat, w2, b2


def _fast_2d(x, weight, bias):
    d0, d1 = x.shape
    b_rows = _GROUP
    for cand in (32, 16, 8):
        if d0 % cand == 0 and cand * d1 <= 32768:
            b_rows = cand
            break
    tq = b_rows * d1 // _GROUP
    s_mat, w2, b2 = _block_diag_mats(weight, bias)
    x4 = x.astype(jnp.float32).reshape(d0 // b_rows, _GROUP, tq)
    return pl.pallas_call(
        _fast_kernel,
        out_shape=jax.ShapeDtypeStruct((d0, d1, _OUT_SIZE), jnp.float32),
        grid=(d0 // b_rows,),
        in_specs=[
            pl.BlockSpec((1, _GROUP, tq), lambda i: (i, 0, 0)),
            pl.BlockSpec((_GROUP, _GROUP * _N_BASIS), lambda i: (0, 0)),
            pl.BlockSpec((_GROUP * _N_BASIS, 128), lambda i: (0, 0)),
            pl.BlockSpec((1, 128), lambda i: (0, 0)),
        ],
        out_specs=pl.BlockSpec((b_rows, d1, _OUT_SIZE), lambda i: (i, 0, 0)),
        compiler_params=pltpu.CompilerParams(
            dimension_semantics=("parallel",)),
    )(x4, s_mat, w2, b2)


def _generic(x, weight, bias):
    orig_shape = x.shape
    x_flat = x.reshape(-1).astype(jnp.float32)
    n = x_flat.shape[0]
    row_align = 128 * _GROUP
    n_ceil = _round_up(max(n, 1), row_align)
    tn = min(16384, n_ceil)
    tile_rows = tn // _GROUP
    n_pad = _round_up(max(n, 1), tn)
    rows_pad = n_pad // _GROUP
    s_mat, w2, b2 = _block_diag_mats(weight, bias)
    xt = jnp.pad(x_flat, (0, n_pad - n)).reshape(rows_pad, _GROUP).T
    out2 = pl.pallas_call(
        _generic_kernel,
        out_shape=jax.ShapeDtypeStruct((rows_pad, 128), jnp.float32),
        grid=(n_pad // tn,),
        in_specs=[
            pl.BlockSpec((_GROUP, tile_rows), lambda i: (0, i)),
            pl.BlockSpec((_GROUP, _GROUP * _N_BASIS), lambda i: (0, 0)),
            pl.BlockSpec((_GROUP * _N_BASIS, 128), lambda i: (0, 0)),
            pl.BlockSpec((1, 128), lambda i: (0, 0)),
        ],
        out_specs=pl.BlockSpec((tile_rows, 128), lambda i: (i, 0)),
        compiler_params=pltpu.CompilerParams(
            dimension_semantics=("parallel",)),
    )(xt, s_mat, w2, b2)
    out = out2.reshape(n_pad, _OUT_SIZE)[:n]
    return out.reshape(*orig_shape, _OUT_SIZE)


@jax.jit
def kernel(x, weight, bias):
    if (x.ndim == 2 and x.shape[0] % _GROUP == 0
            and x.shape[1] % 128 == 0 and x.size >= 16384):
        return _fast_2d(x, weight, bias)
    return _generic(x, weight, bias)
